# baseline (speedup 1.0000x reference)
import functools

import jax
import jax.numpy as jnp
from jax import lax
from jax.experimental import pallas as pl
from jax.experimental.pallas import tpu as pltpu

_B, _N, _DIN, _DOUT = 4, 4096, 64, 128
_NQ, _NG = 1024, 24
_ROWS, _LANES = 32, 128
_QROWS = 8
_BIG = 1e10

_R = _B * _NQ * _NG
_QT = 32
_RT = _QT * _NG
_NT = _R // _RT
_TPB = _NT // _B
_MDF = _NQ * _NG * _DIN
_HIGH = lax.Precision.HIGHEST


def _fps_kernel(zx_ref, zy_ref, zz_ref, idx_ref, cx_ref, cy_ref, cz_ref):
    zx = zx_ref[0]
    zy = zy_ref[0]
    zz = zz_ref[0]
    fi = (lax.broadcasted_iota(jnp.int32, (_ROWS, _LANES), 0) * _LANES
          + lax.broadcasted_iota(jnp.int32, (_ROWS, _LANES), 1))
    qi = (lax.broadcasted_iota(jnp.int32, (_QROWS, _LANES), 0) * _LANES
          + lax.broadcasted_iota(jnp.int32, (_QROWS, _LANES), 1))

    lx0 = zx_ref[0, 0, 0]
    ly0 = zy_ref[0, 0, 0]
    lz0 = zz_ref[0, 0, 0]
    dists0 = jnp.full((_ROWS, _LANES), _BIG, jnp.float32)
    acc_i0 = jnp.zeros((_QROWS, _LANES), jnp.int32)
    acc_x0 = jnp.where(qi == 0, lx0, 0.0).astype(jnp.float32)
    acc_y0 = jnp.where(qi == 0, ly0, 0.0).astype(jnp.float32)
    acc_z0 = jnp.where(qi == 0, lz0, 0.0).astype(jnp.float32)

    def body(i, c):
        dists, lx, ly, lz, acc_i, acc_x, acc_y, acc_z = c
        dx = zx - lx
        dy = zy - ly
        dz = zz - lz
        d = (dx * dx + dy * dy) + dz * dz
        dists = jnp.minimum(dists, d)
        m = jnp.max(dists)
        sel = jnp.min(jnp.where(dists == m, fi, jnp.int32(2**30)))
        hit = fi == sel
        nlx = jnp.sum(jnp.where(hit, zx, 0.0))
        nly = jnp.sum(jnp.where(hit, zy, 0.0))
        nlz = jnp.sum(jnp.where(hit, zz, 0.0))
        at_i = qi == i
        acc_i = jnp.where(at_i, sel, acc_i)
        acc_x = jnp.where(at_i, nlx, acc_x)
        acc_y = jnp.where(at_i, nly, acc_y)
        acc_z = jnp.where(at_i, nlz, acc_z)
        return (dists, nlx, nly, nlz, acc_i, acc_x, acc_y, acc_z)

    c = lax.fori_loop(
        1, _NQ, body,
        (dists0, lx0, ly0, lz0, acc_i0, acc_x0, acc_y0, acc_z0))
    idx_ref[0] = c[4]
    cx_ref[0] = c[5]
    cy_ref[0] = c[6]
    cz_ref[0] = c[7]


def _run_fps(z):
    zr = z.reshape(_B, _ROWS, _LANES, 3)
    zx = zr[..., 0]
    zy = zr[..., 1]
    zz = zr[..., 2]
    out_shapes = (
        jax.ShapeDtypeStruct((_B, _QROWS, _LANES), jnp.int32),
        jax.ShapeDtypeStruct((_B, _QROWS, _LANES), jnp.float32),
        jax.ShapeDtypeStruct((_B, _QROWS, _LANES), jnp.float32),
        jax.ShapeDtypeStruct((_B, _QROWS, _LANES), jnp.float32),
    )
    in_spec = pl.BlockSpec((1, _ROWS, _LANES), lambda b: (b, 0, 0))
    out_spec = pl.BlockSpec((1, _QROWS, _LANES), lambda b: (b, 0, 0))
    idx, cx, cy, cz = pl.pallas_call(
        _fps_kernel,
        grid=(_B,),
        in_specs=[in_spec, in_spec, in_spec],
        out_specs=(out_spec, out_spec, out_spec, out_spec),
        out_shape=out_shapes,
    )(zx, zy, zz)
    u_ce = idx.reshape(_B, _NQ)
    z_ce = jnp.stack(
        [cx.reshape(_B, _NQ), cy.reshape(_B, _NQ), cz.reshape(_B, _NQ)],
        axis=-1)
    return u_ce, z_ce


def _expand_ce(xce):
    return jnp.broadcast_to(
        xce[:, None, :], (_QT, _NG, _DIN)).reshape(_RT, _DIN)


def _acc_init(i, acc_ref):
    @pl.when(i == 0)
    def _():
        acc_ref[...] = jnp.zeros((8, 128), jnp.float32)


def _chan_stats(y):
    s = jnp.sum(y, axis=0, keepdims=True)
    ss = jnp.sum(y * y, axis=0, keepdims=True)
    return jnp.concatenate([s, ss, jnp.zeros((6, 128), jnp.float32)], axis=0)


def _bn_coefs(st_ref, g_ref, b_ref):
    mean = st_ref[0:1, :] / _R
    var = st_ref[1:2, :] / _R - mean * mean
    rstd = lax.rsqrt(var + 1e-5)
    scale = g_ref[0:1, :] * rstd
    shift = b_ref[0:1, :] - mean * scale
    return scale, shift


def _p0_kernel(xne_ref, xce_ref, acc_ref):
    i = pl.program_id(0)
    b = i // _TPB
    dff = xne_ref[...] - _expand_ce(xce_ref[...])
    s = jnp.sum(dff)
    ss = jnp.sum(dff * dff)
    row = lax.broadcasted_iota(jnp.int32, (8, 128), 0)
    upd = (jnp.where(row == b, s, 0.0)
           + jnp.where(row == 4 + b, ss, 0.0)).astype(jnp.float32)
    _acc_init(i, acc_ref)
    acc_ref[...] += upd


def _p1_kernel(xne_ref, xce_ref, al_ref, be_ref, w_ref, st0_ref,
               y_ref, acc_ref):
    i = pl.program_id(0)
    b = i // _TPB
    xcee = _expand_ce(xce_ref[...])
    dff = xne_ref[...] - xcee
    st0 = st0_ref[...]
    row = lax.broadcasted_iota(jnp.int32, (8, 128), 0)
    s = jnp.sum(jnp.where(row == b, st0, 0.0)) / 128.0
    ss = jnp.sum(jnp.where(row == 4 + b, st0, 0.0)) / 128.0
    m = float(_MDF)
    var = (ss - s * s / m) / (m - 1.0)
    std = jnp.sqrt(var)
    dn = al_ref[0:1, :] * dff / (std + 1e-5) + be_ref[0:1, :]
    y = (jnp.dot(dn, w_ref[0:_DIN, :], precision=_HIGH,
                 preferred_element_type=jnp.float32)
         + jnp.dot(xcee, w_ref[_DIN:, :], precision=_HIGH,
                   preferred_element_type=jnp.float32))
    y_ref[...] = y
    _acc_init(i, acc_ref)
    acc_ref[...] += _chan_stats(y)


def _p2_kernel(yp_ref, st_ref, g_ref, b_ref, w_ref, y_ref, acc_ref):
    i = pl.program_id(0)
    scale, shift = _bn_coefs(st_ref, g_ref, b_ref)
    a = jax.nn.relu(yp_ref[...] * scale + shift)
    y = jnp.dot(a, w_ref[...], precision=_HIGH,
                preferred_element_type=jnp.float32)
    y_ref[...] = y
    _acc_init(i, acc_ref)
    acc_ref[...] += _chan_stats(y)


def _p4_kernel(y2_ref, y0_ref, st2_ref, g2_ref, b2_ref,
               st0_ref, g0_ref, b0_ref, w_ref, y_ref, h1_ref, acc_ref):
    i = pl.program_id(0)
    sc0, sh0 = _bn_coefs(st0_ref, g0_ref, b0_ref)
    a0 = jax.nn.relu(y0_ref[...] * sc0 + sh0)
    sc2, sh2 = _bn_coefs(st2_ref, g2_ref, b2_ref)
    h1 = jax.nn.relu(y2_ref[...] * sc2 + sh2 + a0)
    h1_ref[...] = h1
    y = jnp.dot(h1, w_ref[...], precision=_HIGH,
                preferred_element_type=jnp.float32)
    y_ref[...] = y
    _acc_init(i, acc_ref)
    acc_ref[...] += _chan_stats(y)


def _p6_kernel(y4_ref, h1_ref, st4_ref, g4_ref, b4_ref, out_ref):
    sc4, sh4 = _bn_coefs(st4_ref, g4_ref, b4_ref)
    h2 = jax.nn.relu(y4_ref[...] * sc4 + sh4 + h1_ref[...])
    out_ref[...] = jnp.max(h2.reshape(_QT, _NG, _DOUT), axis=1)


def _row_spec(lanes):
    return pl.BlockSpec((_RT, lanes), lambda i: (i, 0))


_CE_SPEC = pl.BlockSpec((_QT, _DIN), lambda i: (i, 0))
_CONST8 = pl.BlockSpec((8, 128), lambda i: (0, 0))
_ACC_SPEC = pl.BlockSpec((8, 128), lambda i: (0, 0))
_ST8 = jax.ShapeDtypeStruct((8, 128), jnp.float32)


def _bcast8(v):
    return jnp.broadcast_to(v.reshape(1, -1), (8, v.size)).astype(jnp.float32)


def _run_mlp(x_ne, x_ce, alpha, beta, W0, g0, b0, blocks):
    (W1a, g1a, b1a, W2a, g2a, b2a), (W1b, g1b, b1b, W2b, g2b, b2b) = blocks
    grid = (_NT,)
    w_spec = pl.BlockSpec((_DOUT, _DOUT), lambda i: (0, 0))
    w0_spec = pl.BlockSpec((2 * _DIN, _DOUT), lambda i: (0, 0))
    ab_spec = pl.BlockSpec((8, _DIN), lambda i: (0, 0))
    yshape = jax.ShapeDtypeStruct((_R, _DOUT), jnp.float32)

    st_d = pl.pallas_call(
        _p0_kernel, grid=grid,
        in_specs=[_row_spec(_DIN), _CE_SPEC],
        out_specs=_ACC_SPEC, out_shape=_ST8,
    )(x_ne, x_ce)

    y0, st0 = pl.pallas_call(
        _p1_kernel, grid=grid,
        in_specs=[_row_spec(_DIN), _CE_SPEC, ab_spec, ab_spec, w0_spec,
                  _CONST8],
        out_specs=(_row_spec(_DOUT), _ACC_SPEC),
        out_shape=(yshape, _ST8),
    )(x_ne, x_ce, _bcast8(alpha), _bcast8(beta), W0, st_d)

    def generic(yp, st, g, b, w):
        return pl.pallas_call(
            _p2_kernel, grid=grid,
            in_specs=[_row_spec(_DOUT), _CONST8, _CONST8, _CONST8, w_spec],
            out_specs=(_row_spec(_DOUT), _ACC_SPEC),
            out_shape=(yshape, _ST8),
        )(yp, st, _bcast8(g), _bcast8(b), w)

    y1, st1 = generic(y0, st0, g0, b0, W1a)
    y2, st2 = generic(y1, st1, g1a, b1a, W2a)

    y3, h1, st3 = pl.pallas_call(
        _p4_kernel, grid=grid,
        in_specs=[_row_spec(_DOUT), _row_spec(_DOUT), _CONST8, _CONST8,
                  _CONST8, _CONST8, _CONST8, _CONST8, w_spec],
        out_specs=(_row_spec(_DOUT), _row_spec(_DOUT), _ACC_SPEC),
        out_shape=(yshape, yshape, _ST8),
    )(y2, y0, st2, _bcast8(g2a), _bcast8(b2a),
      st0, _bcast8(g0), _bcast8(b0), W1b)

    y4, st4 = generic(y3, st3, g1b, b1b, W2b)

    out = pl.pallas_call(
        _p6_kernel, grid=grid,
        in_specs=[_row_spec(_DOUT), _row_spec(_DOUT), _CONST8, _CONST8,
                  _CONST8],
        out_specs=pl.BlockSpec((_QT, _DOUT), lambda i: (i, 0)),
        out_shape=jax.ShapeDtypeStruct((_B * _NQ, _DOUT), jnp.float32),
    )(y4, h1, st4, _bcast8(g2b), _bcast8(b2b))
    return out.reshape(_B, _NQ, _DOUT)


def kernel(x, z, alpha, beta, W0, g0, b0, W1_0, g1_0, b1_0, W2_0, g2_0, b2_0,
           W1_1, g1_1, b1_1, W2_1, g2_1, b2_1):
    u_ce, z_ce = _run_fps(z)

    d = jnp.sum((z_ce[:, :, None, :] - z[:, None, :, :]) ** 2, axis=-1)
    _, u_ne = lax.top_k(-d, _NG)
    gather = jax.vmap(lambda xb, ib: xb[ib])
    x_ce = gather(x, u_ce).reshape(_B * _NQ, _DIN)
    x_ne = gather(x, u_ne).reshape(_R, _DIN)

    blocks = ((W1_0, g1_0, b1_0, W2_0, g2_0, b2_0),
              (W1_1, g1_1, b1_1, W2_1, g2_1, b2_1))
    x_out = _run_mlp(x_ne, x_ce, alpha, beta, W0, g0, b0, blocks)
    return x_out, z_ce


# speedup vs baseline: 1.0494x; 1.0494x over previous
import functools

import jax
import jax.numpy as jnp
from jax import lax
from jax.experimental import pallas as pl
from jax.experimental.pallas import tpu as pltpu

_B, _N, _DIN, _DOUT = 4, 4096, 64, 128
_NQ, _NG = 1024, 24
_ROWS, _LANES = 32, 128
_QROWS = 8
_BIG = 1e10

_R = _B * _NQ * _NG
_QT = 256
_RT = _QT * _NG
_NT = _R // _RT
_TPB = _NT // _B
_MDF = _NQ * _NG * _DIN
_HIGH = lax.Precision.HIGHEST


def _fps_kernel(zx_ref, zy_ref, zz_ref, idx_ref, cx_ref, cy_ref, cz_ref):
    zx = zx_ref[0]
    zy = zy_ref[0]
    zz = zz_ref[0]
    fi = (lax.broadcasted_iota(jnp.int32, (_ROWS, _LANES), 0) * _LANES
          + lax.broadcasted_iota(jnp.int32, (_ROWS, _LANES), 1))
    qi = (lax.broadcasted_iota(jnp.int32, (_QROWS, _LANES), 0) * _LANES
          + lax.broadcasted_iota(jnp.int32, (_QROWS, _LANES), 1))

    lx0 = zx_ref[0, 0, 0]
    ly0 = zy_ref[0, 0, 0]
    lz0 = zz_ref[0, 0, 0]
    dists0 = jnp.full((_ROWS, _LANES), _BIG, jnp.float32)
    acc_i0 = jnp.zeros((_QROWS, _LANES), jnp.int32)
    acc_x0 = jnp.where(qi == 0, lx0, 0.0).astype(jnp.float32)
    acc_y0 = jnp.where(qi == 0, ly0, 0.0).astype(jnp.float32)
    acc_z0 = jnp.where(qi == 0, lz0, 0.0).astype(jnp.float32)

    def body(i, c):
        dists, lx, ly, lz, acc_i, acc_x, acc_y, acc_z = c
        dx = zx - lx
        dy = zy - ly
        dz = zz - lz
        d = (dx * dx + dy * dy) + dz * dz
        dists = jnp.minimum(dists, d)
        m = jnp.max(dists)
        sel = jnp.min(jnp.where(dists == m, fi, jnp.int32(2**30)))
        hit = fi == sel
        nlx = jnp.sum(jnp.where(hit, zx, 0.0))
        nly = jnp.sum(jnp.where(hit, zy, 0.0))
        nlz = jnp.sum(jnp.where(hit, zz, 0.0))
        at_i = qi == i
        acc_i = jnp.where(at_i, sel, acc_i)
        acc_x = jnp.where(at_i, nlx, acc_x)
        acc_y = jnp.where(at_i, nly, acc_y)
        acc_z = jnp.where(at_i, nlz, acc_z)
        return (dists, nlx, nly, nlz, acc_i, acc_x, acc_y, acc_z)

    c = lax.fori_loop(
        1, _NQ, body,
        (dists0, lx0, ly0, lz0, acc_i0, acc_x0, acc_y0, acc_z0))
    idx_ref[0] = c[4]
    cx_ref[0] = c[5]
    cy_ref[0] = c[6]
    cz_ref[0] = c[7]


def _run_fps(z):
    zr = z.reshape(_B, _ROWS, _LANES, 3)
    zx = zr[..., 0]
    zy = zr[..., 1]
    zz = zr[..., 2]
    out_shapes = (
        jax.ShapeDtypeStruct((_B, _QROWS, _LANES), jnp.int32),
        jax.ShapeDtypeStruct((_B, _QROWS, _LANES), jnp.float32),
        jax.ShapeDtypeStruct((_B, _QROWS, _LANES), jnp.float32),
        jax.ShapeDtypeStruct((_B, _QROWS, _LANES), jnp.float32),
    )
    in_spec = pl.BlockSpec((1, _ROWS, _LANES), lambda b: (b, 0, 0))
    out_spec = pl.BlockSpec((1, _QROWS, _LANES), lambda b: (b, 0, 0))
    idx, cx, cy, cz = pl.pallas_call(
        _fps_kernel,
        grid=(_B,),
        in_specs=[in_spec, in_spec, in_spec],
        out_specs=(out_spec, out_spec, out_spec, out_spec),
        out_shape=out_shapes,
    )(zx, zy, zz)
    u_ce = idx.reshape(_B, _NQ)
    z_ce = jnp.stack(
        [cx.reshape(_B, _NQ), cy.reshape(_B, _NQ), cz.reshape(_B, _NQ)],
        axis=-1)
    return u_ce, z_ce


def _expand_ce(xce):
    return jnp.broadcast_to(
        xce[:, None, :], (_QT, _NG, _DIN)).reshape(_RT, _DIN)


def _acc_init(i, acc_ref):
    @pl.when(i == 0)
    def _():
        acc_ref[...] = jnp.zeros((8, 128), jnp.float32)


def _chan_stats(y):
    s = jnp.sum(y, axis=0, keepdims=True)
    ss = jnp.sum(y * y, axis=0, keepdims=True)
    return jnp.concatenate([s, ss, jnp.zeros((6, 128), jnp.float32)], axis=0)


def _bn_coefs(st_ref, g_ref, b_ref):
    mean = st_ref[0:1, :] / _R
    var = st_ref[1:2, :] / _R - mean * mean
    rstd = lax.rsqrt(var + 1e-5)
    scale = g_ref[0:1, :] * rstd
    shift = b_ref[0:1, :] - mean * scale
    return scale, shift


def _p0_kernel(xne_ref, xce_ref, acc_ref):
    i = pl.program_id(0)
    b = i // _TPB
    dff = xne_ref[...] - _expand_ce(xce_ref[...])
    s = jnp.sum(dff)
    ss = jnp.sum(dff * dff)
    row = lax.broadcasted_iota(jnp.int32, (8, 128), 0)
    upd = (jnp.where(row == b, s, 0.0)
           + jnp.where(row == 4 + b, ss, 0.0)).astype(jnp.float32)
    _acc_init(i, acc_ref)
    acc_ref[...] += upd


def _p1_kernel(xne_ref, xce_ref, al_ref, be_ref, w_ref, st0_ref,
               y_ref, acc_ref):
    i = pl.program_id(0)
    b = i // _TPB
    xcee = _expand_ce(xce_ref[...])
    dff = xne_ref[...] - xcee
    st0 = st0_ref[...]
    row = lax.broadcasted_iota(jnp.int32, (8, 128), 0)
    s = jnp.sum(jnp.where(row == b, st0, 0.0)) / 128.0
    ss = jnp.sum(jnp.where(row == 4 + b, st0, 0.0)) / 128.0
    m = float(_MDF)
    var = (ss - s * s / m) / (m - 1.0)
    std = jnp.sqrt(var)
    dn = al_ref[0:1, :] * dff / (std + 1e-5) + be_ref[0:1, :]
    y = (jnp.dot(dn, w_ref[0:_DIN, :], precision=_HIGH,
                 preferred_element_type=jnp.float32)
         + jnp.dot(xcee, w_ref[_DIN:, :], precision=_HIGH,
                   preferred_element_type=jnp.float32))
    y_ref[...] = y
    _acc_init(i, acc_ref)
    acc_ref[...] += _chan_stats(y)


def _p2_kernel(yp_ref, st_ref, g_ref, b_ref, w_ref, y_ref, acc_ref):
    i = pl.program_id(0)
    scale, shift = _bn_coefs(st_ref, g_ref, b_ref)
    a = jax.nn.relu(yp_ref[...] * scale + shift)
    y = jnp.dot(a, w_ref[...], precision=_HIGH,
                preferred_element_type=jnp.float32)
    y_ref[...] = y
    _acc_init(i, acc_ref)
    acc_ref[...] += _chan_stats(y)


def _p4_kernel(y2_ref, y0_ref, st2_ref, g2_ref, b2_ref,
               st0_ref, g0_ref, b0_ref, w_ref, y_ref, h1_ref, acc_ref):
    i = pl.program_id(0)
    sc0, sh0 = _bn_coefs(st0_ref, g0_ref, b0_ref)
    a0 = jax.nn.relu(y0_ref[...] * sc0 + sh0)
    sc2, sh2 = _bn_coefs(st2_ref, g2_ref, b2_ref)
    h1 = jax.nn.relu(y2_ref[...] * sc2 + sh2 + a0)
    h1_ref[...] = h1
    y = jnp.dot(h1, w_ref[...], precision=_HIGH,
                preferred_element_type=jnp.float32)
    y_ref[...] = y
    _acc_init(i, acc_ref)
    acc_ref[...] += _chan_stats(y)


def _p6_kernel(y4_ref, h1_ref, st4_ref, g4_ref, b4_ref, out_ref):
    sc4, sh4 = _bn_coefs(st4_ref, g4_ref, b4_ref)
    h2 = jax.nn.relu(y4_ref[...] * sc4 + sh4 + h1_ref[...])
    out_ref[...] = jnp.max(h2.reshape(_QT, _NG, _DOUT), axis=1)


def _row_spec(lanes):
    return pl.BlockSpec((_RT, lanes), lambda i: (i, 0))


_CE_SPEC = pl.BlockSpec((_QT, _DIN), lambda i: (i, 0))
_CONST8 = pl.BlockSpec((8, 128), lambda i: (0, 0))
_ACC_SPEC = pl.BlockSpec((8, 128), lambda i: (0, 0))
_ST8 = jax.ShapeDtypeStruct((8, 128), jnp.float32)


def _bcast8(v):
    return jnp.broadcast_to(v.reshape(1, -1), (8, v.size)).astype(jnp.float32)


def _run_mlp(x_ne, x_ce, alpha, beta, W0, g0, b0, blocks):
    (W1a, g1a, b1a, W2a, g2a, b2a), (W1b, g1b, b1b, W2b, g2b, b2b) = blocks
    grid = (_NT,)
    w_spec = pl.BlockSpec((_DOUT, _DOUT), lambda i: (0, 0))
    w0_spec = pl.BlockSpec((2 * _DIN, _DOUT), lambda i: (0, 0))
    ab_spec = pl.BlockSpec((8, _DIN), lambda i: (0, 0))
    yshape = jax.ShapeDtypeStruct((_R, _DOUT), jnp.float32)

    st_d = pl.pallas_call(
        _p0_kernel, grid=grid,
        in_specs=[_row_spec(_DIN), _CE_SPEC],
        out_specs=_ACC_SPEC, out_shape=_ST8,
    )(x_ne, x_ce)

    y0, st0 = pl.pallas_call(
        _p1_kernel, grid=grid,
        in_specs=[_row_spec(_DIN), _CE_SPEC, ab_spec, ab_spec, w0_spec,
                  _CONST8],
        out_specs=(_row_spec(_DOUT), _ACC_SPEC),
        out_shape=(yshape, _ST8),
    )(x_ne, x_ce, _bcast8(alpha), _bcast8(beta), W0, st_d)

    def generic(yp, st, g, b, w):
        return pl.pallas_call(
            _p2_kernel, grid=grid,
            in_specs=[_row_spec(_DOUT), _CONST8, _CONST8, _CONST8, w_spec],
            out_specs=(_row_spec(_DOUT), _ACC_SPEC),
            out_shape=(yshape, _ST8),
        )(yp, st, _bcast8(g), _bcast8(b), w)

    y1, st1 = generic(y0, st0, g0, b0, W1a)
    y2, st2 = generic(y1, st1, g1a, b1a, W2a)

    y3, h1, st3 = pl.pallas_call(
        _p4_kernel, grid=grid,
        in_specs=[_row_spec(_DOUT), _row_spec(_DOUT), _CONST8, _CONST8,
                  _CONST8, _CONST8, _CONST8, _CONST8, w_spec],
        out_specs=(_row_spec(_DOUT), _row_spec(_DOUT), _ACC_SPEC),
        out_shape=(yshape, yshape, _ST8),
    )(y2, y0, st2, _bcast8(g2a), _bcast8(b2a),
      st0, _bcast8(g0), _bcast8(b0), W1b)

    y4, st4 = generic(y3, st3, g1b, b1b, W2b)

    out = pl.pallas_call(
        _p6_kernel, grid=grid,
        in_specs=[_row_spec(_DOUT), _row_spec(_DOUT), _CONST8, _CONST8,
                  _CONST8],
        out_specs=pl.BlockSpec((_QT, _DOUT), lambda i: (i, 0)),
        out_shape=jax.ShapeDtypeStruct((_B * _NQ, _DOUT), jnp.float32),
    )(y4, h1, st4, _bcast8(g2b), _bcast8(b2b))
    return out.reshape(_B, _NQ, _DOUT)


def kernel(x, z, alpha, beta, W0, g0, b0, W1_0, g1_0, b1_0, W2_0, g2_0, b2_0,
           W1_1, g1_1, b1_1, W2_1, g2_1, b2_1):
    u_ce, z_ce = _run_fps(z)

    d = jnp.sum((z_ce[:, :, None, :] - z[:, None, :, :]) ** 2, axis=-1)
    _, u_ne = lax.top_k(-d, _NG)
    gather = jax.vmap(lambda xb, ib: xb[ib])
    x_ce = gather(x, u_ce).reshape(_B * _NQ, _DIN)
    x_ne = gather(x, u_ne).reshape(_R, _DIN)

    blocks = ((W1_0, g1_0, b1_0, W2_0, g2_0, b2_0),
              (W1_1, g1_1, b1_1, W2_1, g2_1, b2_1))
    x_out = _run_mlp(x_ne, x_ce, alpha, beta, W0, g0, b0, blocks)
    return x_out, z_ce


# speedup vs baseline: 1.3258x; 1.2634x over previous
import functools

import jax
import jax.numpy as jnp
from jax import lax
from jax.experimental import pallas as pl
from jax.experimental.pallas import tpu as pltpu

_B, _N, _DIN, _DOUT = 4, 4096, 64, 128
_NQ, _NG = 1024, 24
_ROWS, _LANES = 32, 128
_QROWS = 8
_BIG = 1e10

_R = _B * _NQ * _NG
_QT = 256
_RT = _QT * _NG
_NT = _R // _RT
_TPB = _NT // _B
_MDF = _NQ * _NG * _DIN
_HIGH = lax.Precision.HIGHEST


def _fps_kernel(zx_ref, zy_ref, zz_ref, idx_ref, cx_ref, cy_ref, cz_ref):
    zx = zx_ref[0]
    zy = zy_ref[0]
    zz = zz_ref[0]
    fi = (lax.broadcasted_iota(jnp.int32, (_ROWS, _LANES), 0) * _LANES
          + lax.broadcasted_iota(jnp.int32, (_ROWS, _LANES), 1))
    qi = (lax.broadcasted_iota(jnp.int32, (_QROWS, _LANES), 0) * _LANES
          + lax.broadcasted_iota(jnp.int32, (_QROWS, _LANES), 1))

    lx0 = zx_ref[0, 0, 0]
    ly0 = zy_ref[0, 0, 0]
    lz0 = zz_ref[0, 0, 0]
    dists0 = jnp.full((_ROWS, _LANES), _BIG, jnp.float32)
    acc_i0 = jnp.zeros((_QROWS, _LANES), jnp.int32)
    acc_x0 = jnp.where(qi == 0, lx0, 0.0).astype(jnp.float32)
    acc_y0 = jnp.where(qi == 0, ly0, 0.0).astype(jnp.float32)
    acc_z0 = jnp.where(qi == 0, lz0, 0.0).astype(jnp.float32)

    def body(i, c):
        dists, lx, ly, lz, acc_i, acc_x, acc_y, acc_z = c
        dx = zx - lx
        dy = zy - ly
        dz = zz - lz
        d = (dx * dx + dy * dy) + dz * dz
        dists = jnp.minimum(dists, d)
        m = jnp.max(dists)
        sel = jnp.min(jnp.where(dists == m, fi, jnp.int32(2**30)))
        hit = fi == sel
        nlx = jnp.sum(jnp.where(hit, zx, 0.0))
        nly = jnp.sum(jnp.where(hit, zy, 0.0))
        nlz = jnp.sum(jnp.where(hit, zz, 0.0))
        at_i = qi == i
        acc_i = jnp.where(at_i, sel, acc_i)
        acc_x = jnp.where(at_i, nlx, acc_x)
        acc_y = jnp.where(at_i, nly, acc_y)
        acc_z = jnp.where(at_i, nlz, acc_z)
        return (dists, nlx, nly, nlz, acc_i, acc_x, acc_y, acc_z)

    c = lax.fori_loop(
        1, _NQ, body,
        (dists0, lx0, ly0, lz0, acc_i0, acc_x0, acc_y0, acc_z0))
    idx_ref[0] = c[4]
    cx_ref[0] = c[5]
    cy_ref[0] = c[6]
    cz_ref[0] = c[7]


def _run_fps(z):
    zr = z.reshape(_B, _ROWS, _LANES, 3)
    zx = zr[..., 0]
    zy = zr[..., 1]
    zz = zr[..., 2]
    out_shapes = (
        jax.ShapeDtypeStruct((_B, _QROWS, _LANES), jnp.int32),
        jax.ShapeDtypeStruct((_B, _QROWS, _LANES), jnp.float32),
        jax.ShapeDtypeStruct((_B, _QROWS, _LANES), jnp.float32),
        jax.ShapeDtypeStruct((_B, _QROWS, _LANES), jnp.float32),
    )
    in_spec = pl.BlockSpec((1, _ROWS, _LANES), lambda b: (b, 0, 0))
    out_spec = pl.BlockSpec((1, _QROWS, _LANES), lambda b: (b, 0, 0))
    idx, cx, cy, cz = pl.pallas_call(
        _fps_kernel,
        grid=(_B,),
        in_specs=[in_spec, in_spec, in_spec],
        out_specs=(out_spec, out_spec, out_spec, out_spec),
        out_shape=out_shapes,
    )(zx, zy, zz)
    u_ce = idx.reshape(_B, _NQ)
    z_ce = jnp.stack(
        [cx.reshape(_B, _NQ), cy.reshape(_B, _NQ), cz.reshape(_B, _NQ)],
        axis=-1)
    return u_ce, z_ce


_NW = 32
_CH = 128
_NEPW = _R // _NW
_NCH = _NEPW // _CH
_CEPW = (_B * _NQ) // _NW


def _run_gather(x, u_ne, u_ce):
    from jax.experimental.pallas import tpu_sc as plsc

    xf = jnp.pad(x.reshape(_B * _N, _DIN), ((0, 0), (0, 128 - _DIN)))
    boff = (jnp.arange(_B, dtype=jnp.int32) * _N)
    ine = (u_ne.astype(jnp.int32) + boff[:, None, None]).reshape(_R)
    ice = (u_ce.astype(jnp.int32) + boff[:, None]).reshape(_B * _NQ)

    mesh = plsc.VectorSubcoreMesh(core_axis_name="c", subcore_axis_name="s")

    @functools.partial(
        pl.kernel, mesh=mesh,
        out_type=(jax.ShapeDtypeStruct((_R, 128), jnp.float32),
                  jax.ShapeDtypeStruct((_B * _NQ, 128), jnp.float32)),
        scratch_types=[pltpu.VMEM((_CH,), jnp.int32),
                       pltpu.VMEM((_CH, 128), jnp.float32),
                       pltpu.SemaphoreType.DMA],
    )
    def gk(xf_hbm, ine_hbm, ice_hbm, one_hbm, oce_hbm, idx_v, rows_v, sem):
        wid = lax.axis_index("s") * 2 + lax.axis_index("c")

        def body(c, carry):
            base = wid * _NEPW + c * _CH
            pltpu.sync_copy(ine_hbm.at[pl.ds(base, _CH)], idx_v)
            pltpu.async_copy(xf_hbm.at[idx_v], rows_v, sem).wait()
            pltpu.sync_copy(rows_v, one_hbm.at[pl.ds(base, _CH)])
            return carry

        lax.fori_loop(0, _NCH, body, 0)
        cbase = wid * _CEPW
        pltpu.sync_copy(ice_hbm.at[pl.ds(cbase, _CEPW)], idx_v)
        pltpu.async_copy(xf_hbm.at[idx_v], rows_v, sem).wait()
        pltpu.sync_copy(rows_v, oce_hbm.at[pl.ds(cbase, _CEPW)])

    return gk(xf, ine, ice)


def _expand_ce(xce):
    return jnp.broadcast_to(
        xce[:, None, :], (_QT, _NG, _DIN)).reshape(_RT, _DIN)


def _acc_init(i, acc_ref):
    @pl.when(i == 0)
    def _():
        acc_ref[...] = jnp.zeros((8, 128), jnp.float32)


def _chan_stats(y):
    s = jnp.sum(y, axis=0, keepdims=True)
    ss = jnp.sum(y * y, axis=0, keepdims=True)
    return jnp.concatenate([s, ss, jnp.zeros((6, 128), jnp.float32)], axis=0)


def _bn_coefs(st_ref, g_ref, b_ref):
    mean = st_ref[0:1, :] / _R
    var = st_ref[1:2, :] / _R - mean * mean
    rstd = lax.rsqrt(var + 1e-5)
    scale = g_ref[0:1, :] * rstd
    shift = b_ref[0:1, :] - mean * scale
    return scale, shift


def _p0_kernel(xne_ref, xce_ref, acc_ref):
    i = pl.program_id(0)
    b = i // _TPB
    dff = xne_ref[:, 0:_DIN] - _expand_ce(xce_ref[:, 0:_DIN])
    s = jnp.sum(dff)
    ss = jnp.sum(dff * dff)
    row = lax.broadcasted_iota(jnp.int32, (8, 128), 0)
    upd = (jnp.where(row == b, s, 0.0)
           + jnp.where(row == 4 + b, ss, 0.0)).astype(jnp.float32)
    _acc_init(i, acc_ref)
    acc_ref[...] += upd


def _p1_kernel(xne_ref, xce_ref, al_ref, be_ref, w_ref, st0_ref,
               y_ref, acc_ref):
    i = pl.program_id(0)
    b = i // _TPB
    xcee = _expand_ce(xce_ref[:, 0:_DIN])
    dff = xne_ref[:, 0:_DIN] - xcee
    st0 = st0_ref[...]
    row = lax.broadcasted_iota(jnp.int32, (8, 128), 0)
    s = jnp.sum(jnp.where(row == b, st0, 0.0)) / 128.0
    ss = jnp.sum(jnp.where(row == 4 + b, st0, 0.0)) / 128.0
    m = float(_MDF)
    var = (ss - s * s / m) / (m - 1.0)
    std = jnp.sqrt(var)
    dn = al_ref[0:1, :] * dff / (std + 1e-5) + be_ref[0:1, :]
    y = (jnp.dot(dn, w_ref[0:_DIN, :], precision=_HIGH,
                 preferred_element_type=jnp.float32)
         + jnp.dot(xcee, w_ref[_DIN:, :], precision=_HIGH,
                   preferred_element_type=jnp.float32))
    y_ref[...] = y
    _acc_init(i, acc_ref)
    acc_ref[...] += _chan_stats(y)


def _p2_kernel(yp_ref, st_ref, g_ref, b_ref, w_ref, y_ref, acc_ref):
    i = pl.program_id(0)
    scale, shift = _bn_coefs(st_ref, g_ref, b_ref)
    a = jax.nn.relu(yp_ref[...] * scale + shift)
    y = jnp.dot(a, w_ref[...], precision=_HIGH,
                preferred_element_type=jnp.float32)
    y_ref[...] = y
    _acc_init(i, acc_ref)
    acc_ref[...] += _chan_stats(y)


def _p4_kernel(y2_ref, y0_ref, st2_ref, g2_ref, b2_ref,
               st0_ref, g0_ref, b0_ref, w_ref, y_ref, h1_ref, acc_ref):
    i = pl.program_id(0)
    sc0, sh0 = _bn_coefs(st0_ref, g0_ref, b0_ref)
    a0 = jax.nn.relu(y0_ref[...] * sc0 + sh0)
    sc2, sh2 = _bn_coefs(st2_ref, g2_ref, b2_ref)
    h1 = jax.nn.relu(y2_ref[...] * sc2 + sh2 + a0)
    h1_ref[...] = h1
    y = jnp.dot(h1, w_ref[...], precision=_HIGH,
                preferred_element_type=jnp.float32)
    y_ref[...] = y
    _acc_init(i, acc_ref)
    acc_ref[...] += _chan_stats(y)


def _p6_kernel(y4_ref, h1_ref, st4_ref, g4_ref, b4_ref, out_ref):
    sc4, sh4 = _bn_coefs(st4_ref, g4_ref, b4_ref)
    h2 = jax.nn.relu(y4_ref[...] * sc4 + sh4 + h1_ref[...])
    out_ref[...] = jnp.max(h2.reshape(_QT, _NG, _DOUT), axis=1)


def _row_spec(lanes):
    return pl.BlockSpec((_RT, lanes), lambda i: (i, 0))


_CE_SPEC = pl.BlockSpec((_QT, 128), lambda i: (i, 0))
_CONST8 = pl.BlockSpec((8, 128), lambda i: (0, 0))
_ACC_SPEC = pl.BlockSpec((8, 128), lambda i: (0, 0))
_ST8 = jax.ShapeDtypeStruct((8, 128), jnp.float32)


def _bcast8(v):
    return jnp.broadcast_to(v.reshape(1, -1), (8, v.size)).astype(jnp.float32)


def _run_mlp(x_ne, x_ce, alpha, beta, W0, g0, b0, blocks):
    (W1a, g1a, b1a, W2a, g2a, b2a), (W1b, g1b, b1b, W2b, g2b, b2b) = blocks
    grid = (_NT,)
    w_spec = pl.BlockSpec((_DOUT, _DOUT), lambda i: (0, 0))
    w0_spec = pl.BlockSpec((2 * _DIN, _DOUT), lambda i: (0, 0))
    ab_spec = pl.BlockSpec((8, _DIN), lambda i: (0, 0))
    yshape = jax.ShapeDtypeStruct((_R, _DOUT), jnp.float32)

    st_d = pl.pallas_call(
        _p0_kernel, grid=grid,
        in_specs=[_row_spec(128), _CE_SPEC],
        out_specs=_ACC_SPEC, out_shape=_ST8,
    )(x_ne, x_ce)

    y0, st0 = pl.pallas_call(
        _p1_kernel, grid=grid,
        in_specs=[_row_spec(128), _CE_SPEC, ab_spec, ab_spec, w0_spec,
                  _CONST8],
        out_specs=(_row_spec(_DOUT), _ACC_SPEC),
        out_shape=(yshape, _ST8),
    )(x_ne, x_ce, _bcast8(alpha), _bcast8(beta), W0, st_d)

    def generic(yp, st, g, b, w):
        return pl.pallas_call(
            _p2_kernel, grid=grid,
            in_specs=[_row_spec(_DOUT), _CONST8, _CONST8, _CONST8, w_spec],
            out_specs=(_row_spec(_DOUT), _ACC_SPEC),
            out_shape=(yshape, _ST8),
        )(yp, st, _bcast8(g), _bcast8(b), w)

    y1, st1 = generic(y0, st0, g0, b0, W1a)
    y2, st2 = generic(y1, st1, g1a, b1a, W2a)

    y3, h1, st3 = pl.pallas_call(
        _p4_kernel, grid=grid,
        in_specs=[_row_spec(_DOUT), _row_spec(_DOUT), _CONST8, _CONST8,
                  _CONST8, _CONST8, _CONST8, _CONST8, w_spec],
        out_specs=(_row_spec(_DOUT), _row_spec(_DOUT), _ACC_SPEC),
        out_shape=(yshape, yshape, _ST8),
    )(y2, y0, st2, _bcast8(g2a), _bcast8(b2a),
      st0, _bcast8(g0), _bcast8(b0), W1b)

    y4, st4 = generic(y3, st3, g1b, b1b, W2b)

    out = pl.pallas_call(
        _p6_kernel, grid=grid,
        in_specs=[_row_spec(_DOUT), _row_spec(_DOUT), _CONST8, _CONST8,
                  _CONST8],
        out_specs=pl.BlockSpec((_QT, _DOUT), lambda i: (i, 0)),
        out_shape=jax.ShapeDtypeStruct((_B * _NQ, _DOUT), jnp.float32),
    )(y4, h1, st4, _bcast8(g2b), _bcast8(b2b))
    return out.reshape(_B, _NQ, _DOUT)


def kernel(x, z, alpha, beta, W0, g0, b0, W1_0, g1_0, b1_0, W2_0, g2_0, b2_0,
           W1_1, g1_1, b1_1, W2_1, g2_1, b2_1):
    u_ce, z_ce = _run_fps(z)

    d = jnp.sum((z_ce[:, :, None, :] - z[:, None, :, :]) ** 2, axis=-1)
    _, u_ne = lax.top_k(-d, _NG)
    x_ne, x_ce = _run_gather(x, u_ne, u_ce)

    blocks = ((W1_0, g1_0, b1_0, W2_0, g2_0, b2_0),
              (W1_1, g1_1, b1_1, W2_1, g2_1, b2_1))
    x_out = _run_mlp(x_ne, x_ce, alpha, beta, W0, g0, b0, blocks)
    return x_out, z_ce


# speedup vs baseline: 1.3313x; 1.0041x over previous
import functools

import jax
import jax.numpy as jnp
from jax import lax
from jax.experimental import pallas as pl
from jax.experimental.pallas import tpu as pltpu

_B, _N, _DIN, _DOUT = 4, 4096, 64, 128
_NQ, _NG = 1024, 24
_ROWS, _LANES = 32, 128
_QROWS = 8
_BIG = 1e10

_R = _B * _NQ * _NG
_QT = 256
_RT = _QT * _NG
_NT = _R // _RT
_TPB = _NT // _B
_MDF = _NQ * _NG * _DIN
_HIGH = lax.Precision.HIGHEST


def _fps_kernel(zx_ref, zy_ref, zz_ref, idx_ref, cx_ref, cy_ref, cz_ref):
    zx = zx_ref[0]
    zy = zy_ref[0]
    zz = zz_ref[0]
    fi = (lax.broadcasted_iota(jnp.int32, (_ROWS, _LANES), 0) * _LANES
          + lax.broadcasted_iota(jnp.int32, (_ROWS, _LANES), 1))
    qi = (lax.broadcasted_iota(jnp.int32, (_QROWS, _LANES), 0) * _LANES
          + lax.broadcasted_iota(jnp.int32, (_QROWS, _LANES), 1))

    lx0 = zx_ref[0, 0, 0]
    ly0 = zy_ref[0, 0, 0]
    lz0 = zz_ref[0, 0, 0]
    dists0 = jnp.full((_ROWS, _LANES), _BIG, jnp.float32)
    acc_i0 = jnp.zeros((_QROWS, _LANES), jnp.int32)
    acc_x0 = jnp.where(qi == 0, lx0, 0.0).astype(jnp.float32)
    acc_y0 = jnp.where(qi == 0, ly0, 0.0).astype(jnp.float32)
    acc_z0 = jnp.where(qi == 0, lz0, 0.0).astype(jnp.float32)

    def body(i, c):
        dists, lx, ly, lz, acc_i, acc_x, acc_y, acc_z = c
        dx = zx - lx
        dy = zy - ly
        dz = zz - lz
        d = (dx * dx + dy * dy) + dz * dz
        dists = jnp.minimum(dists, d)
        m = jnp.max(dists)
        sel = jnp.min(jnp.where(dists == m, fi, jnp.int32(2**30)))
        hit = fi == sel
        nlx = jnp.sum(jnp.where(hit, zx, 0.0))
        nly = jnp.sum(jnp.where(hit, zy, 0.0))
        nlz = jnp.sum(jnp.where(hit, zz, 0.0))
        at_i = qi == i
        acc_i = jnp.where(at_i, sel, acc_i)
        acc_x = jnp.where(at_i, nlx, acc_x)
        acc_y = jnp.where(at_i, nly, acc_y)
        acc_z = jnp.where(at_i, nlz, acc_z)
        return (dists, nlx, nly, nlz, acc_i, acc_x, acc_y, acc_z)

    c = lax.fori_loop(
        1, _NQ, body,
        (dists0, lx0, ly0, lz0, acc_i0, acc_x0, acc_y0, acc_z0))
    idx_ref[0] = c[4]
    cx_ref[0] = c[5]
    cy_ref[0] = c[6]
    cz_ref[0] = c[7]


def _run_fps(z):
    zr = z.reshape(_B, _ROWS, _LANES, 3)
    zx = zr[..., 0]
    zy = zr[..., 1]
    zz = zr[..., 2]
    out_shapes = (
        jax.ShapeDtypeStruct((_B, _QROWS, _LANES), jnp.int32),
        jax.ShapeDtypeStruct((_B, _QROWS, _LANES), jnp.float32),
        jax.ShapeDtypeStruct((_B, _QROWS, _LANES), jnp.float32),
        jax.ShapeDtypeStruct((_B, _QROWS, _LANES), jnp.float32),
    )
    in_spec = pl.BlockSpec((1, _ROWS, _LANES), lambda b: (b, 0, 0))
    out_spec = pl.BlockSpec((1, _QROWS, _LANES), lambda b: (b, 0, 0))
    idx, cx, cy, cz = pl.pallas_call(
        _fps_kernel,
        grid=(_B,),
        in_specs=[in_spec, in_spec, in_spec],
        out_specs=(out_spec, out_spec, out_spec, out_spec),
        out_shape=out_shapes,
    )(zx, zy, zz)
    u_ce = idx.reshape(_B, _NQ)
    z_ce = jnp.stack(
        [cx.reshape(_B, _NQ), cy.reshape(_B, _NQ), cz.reshape(_B, _NQ)],
        axis=-1)
    return u_ce, z_ce


_QTK = 256


def _dist_kernel(qx_ref, qy_ref, qz_ref, px_ref, py_ref, pz_ref, out_ref):
    qx = qx_ref[0]
    qy = qy_ref[0]
    qz = qz_ref[0]
    px = px_ref[0]
    py = py_ref[0]
    pz = pz_ref[0]
    dx = qx - px
    dy = qy - py
    dz = qz - pz
    out_ref[0] = -((dx * dx + dy * dy) + dz * dz)


def _run_dist(z_ce, z):
    q_col = z_ce[..., None]
    p_row = jnp.swapaxes(z, 1, 2)[:, :, None, :]
    q_spec = pl.BlockSpec((1, _QTK, 1), lambda b, t: (b, t, 0))
    p_spec = pl.BlockSpec((1, 1, _N), lambda b, t: (b, 0, 0))
    return pl.pallas_call(
        _dist_kernel,
        grid=(_B, _NQ // _QTK),
        in_specs=[q_spec, q_spec, q_spec, p_spec, p_spec, p_spec],
        out_specs=pl.BlockSpec((1, _QTK, _N), lambda b, t: (b, t, 0)),
        out_shape=jax.ShapeDtypeStruct((_B, _NQ, _N), jnp.float32),
    )(q_col[:, :, 0], q_col[:, :, 1], q_col[:, :, 2],
      p_row[:, 0], p_row[:, 1], p_row[:, 2])


_NW = 32
_CH = 128
_NEPW = _R // _NW
_NCH = _NEPW // _CH
_CEPW = (_B * _NQ) // _NW


def _run_gather(x, u_ne, u_ce):
    from jax.experimental.pallas import tpu_sc as plsc

    xf = jnp.pad(x.reshape(_B * _N, _DIN), ((0, 0), (0, 128 - _DIN)))
    boff = (jnp.arange(_B, dtype=jnp.int32) * _N)
    ine = (u_ne.astype(jnp.int32) + boff[:, None, None]).reshape(_R)
    ice = (u_ce.astype(jnp.int32) + boff[:, None]).reshape(_B * _NQ)

    mesh = plsc.VectorSubcoreMesh(core_axis_name="c", subcore_axis_name="s")

    @functools.partial(
        pl.kernel, mesh=mesh,
        out_type=(jax.ShapeDtypeStruct((_R, 128), jnp.float32),
                  jax.ShapeDtypeStruct((_B * _NQ, 128), jnp.float32)),
        scratch_types=[pltpu.VMEM((_CH,), jnp.int32),
                       pltpu.VMEM((_CH, 128), jnp.float32),
                       pltpu.SemaphoreType.DMA],
    )
    def gk(xf_hbm, ine_hbm, ice_hbm, one_hbm, oce_hbm, idx_v, rows_v, sem):
        wid = lax.axis_index("s") * 2 + lax.axis_index("c")

        def body(c, carry):
            base = wid * _NEPW + c * _CH
            pltpu.sync_copy(ine_hbm.at[pl.ds(base, _CH)], idx_v)
            pltpu.async_copy(xf_hbm.at[idx_v], rows_v, sem).wait()
            pltpu.sync_copy(rows_v, one_hbm.at[pl.ds(base, _CH)])
            return carry

        lax.fori_loop(0, _NCH, body, 0)
        cbase = wid * _CEPW
        pltpu.sync_copy(ice_hbm.at[pl.ds(cbase, _CEPW)], idx_v)
        pltpu.async_copy(xf_hbm.at[idx_v], rows_v, sem).wait()
        pltpu.sync_copy(rows_v, oce_hbm.at[pl.ds(cbase, _CEPW)])

    return gk(xf, ine, ice)


def _expand_ce(xce):
    return jnp.broadcast_to(
        xce[:, None, :], (_QT, _NG, _DIN)).reshape(_RT, _DIN)


def _acc_init(i, acc_ref):
    @pl.when(i == 0)
    def _():
        acc_ref[...] = jnp.zeros((8, 128), jnp.float32)


def _chan_stats(y):
    s = jnp.sum(y, axis=0, keepdims=True)
    ss = jnp.sum(y * y, axis=0, keepdims=True)
    return jnp.concatenate([s, ss, jnp.zeros((6, 128), jnp.float32)], axis=0)


def _bn_coefs(st_ref, g_ref, b_ref):
    mean = st_ref[0:1, :] / _R
    var = st_ref[1:2, :] / _R - mean * mean
    rstd = lax.rsqrt(var + 1e-5)
    scale = g_ref[0:1, :] * rstd
    shift = b_ref[0:1, :] - mean * scale
    return scale, shift


def _p0_kernel(xne_ref, xce_ref, acc_ref):
    i = pl.program_id(0)
    b = i // _TPB
    dff = xne_ref[:, 0:_DIN] - _expand_ce(xce_ref[:, 0:_DIN])
    s = jnp.sum(dff)
    ss = jnp.sum(dff * dff)
    row = lax.broadcasted_iota(jnp.int32, (8, 128), 0)
    upd = (jnp.where(row == b, s, 0.0)
           + jnp.where(row == 4 + b, ss, 0.0)).astype(jnp.float32)
    _acc_init(i, acc_ref)
    acc_ref[...] += upd


def _p1_kernel(xne_ref, xce_ref, al_ref, be_ref, w_ref, st0_ref,
               y_ref, acc_ref):
    i = pl.program_id(0)
    b = i // _TPB
    xcee = _expand_ce(xce_ref[:, 0:_DIN])
    dff = xne_ref[:, 0:_DIN] - xcee
    st0 = st0_ref[...]
    row = lax.broadcasted_iota(jnp.int32, (8, 128), 0)
    s = jnp.sum(jnp.where(row == b, st0, 0.0)) / 128.0
    ss = jnp.sum(jnp.where(row == 4 + b, st0, 0.0)) / 128.0
    m = float(_MDF)
    var = (ss - s * s / m) / (m - 1.0)
    std = jnp.sqrt(var)
    dn = al_ref[0:1, :] * dff / (std + 1e-5) + be_ref[0:1, :]
    y = (jnp.dot(dn, w_ref[0:_DIN, :], precision=_HIGH,
                 preferred_element_type=jnp.float32)
         + jnp.dot(xcee, w_ref[_DIN:, :], precision=_HIGH,
                   preferred_element_type=jnp.float32))
    y_ref[...] = y
    _acc_init(i, acc_ref)
    acc_ref[...] += _chan_stats(y)


def _p2_kernel(yp_ref, st_ref, g_ref, b_ref, w_ref, y_ref, acc_ref):
    i = pl.program_id(0)
    scale, shift = _bn_coefs(st_ref, g_ref, b_ref)
    a = jax.nn.relu(yp_ref[...] * scale + shift)
    y = jnp.dot(a, w_ref[...], precision=_HIGH,
                preferred_element_type=jnp.float32)
    y_ref[...] = y
    _acc_init(i, acc_ref)
    acc_ref[...] += _chan_stats(y)


def _p4_kernel(y2_ref, y0_ref, st2_ref, g2_ref, b2_ref,
               st0_ref, g0_ref, b0_ref, w_ref, y_ref, h1_ref, acc_ref):
    i = pl.program_id(0)
    sc0, sh0 = _bn_coefs(st0_ref, g0_ref, b0_ref)
    a0 = jax.nn.relu(y0_ref[...] * sc0 + sh0)
    sc2, sh2 = _bn_coefs(st2_ref, g2_ref, b2_ref)
    h1 = jax.nn.relu(y2_ref[...] * sc2 + sh2 + a0)
    h1_ref[...] = h1
    y = jnp.dot(h1, w_ref[...], precision=_HIGH,
                preferred_element_type=jnp.float32)
    y_ref[...] = y
    _acc_init(i, acc_ref)
    acc_ref[...] += _chan_stats(y)


def _p6_kernel(y4_ref, h1_ref, st4_ref, g4_ref, b4_ref, out_ref):
    sc4, sh4 = _bn_coefs(st4_ref, g4_ref, b4_ref)
    h2 = jax.nn.relu(y4_ref[...] * sc4 + sh4 + h1_ref[...])
    out_ref[...] = jnp.max(h2.reshape(_QT, _NG, _DOUT), axis=1)


def _row_spec(lanes):
    return pl.BlockSpec((_RT, lanes), lambda i: (i, 0))


_CE_SPEC = pl.BlockSpec((_QT, 128), lambda i: (i, 0))
_CONST8 = pl.BlockSpec((8, 128), lambda i: (0, 0))
_ACC_SPEC = pl.BlockSpec((8, 128), lambda i: (0, 0))
_ST8 = jax.ShapeDtypeStruct((8, 128), jnp.float32)


def _bcast8(v):
    return jnp.broadcast_to(v.reshape(1, -1), (8, v.size)).astype(jnp.float32)


def _run_mlp(x_ne, x_ce, alpha, beta, W0, g0, b0, blocks):
    (W1a, g1a, b1a, W2a, g2a, b2a), (W1b, g1b, b1b, W2b, g2b, b2b) = blocks
    grid = (_NT,)
    w_spec = pl.BlockSpec((_DOUT, _DOUT), lambda i: (0, 0))
    w0_spec = pl.BlockSpec((2 * _DIN, _DOUT), lambda i: (0, 0))
    ab_spec = pl.BlockSpec((8, _DIN), lambda i: (0, 0))
    yshape = jax.ShapeDtypeStruct((_R, _DOUT), jnp.float32)

    st_d = pl.pallas_call(
        _p0_kernel, grid=grid,
        in_specs=[_row_spec(128), _CE_SPEC],
        out_specs=_ACC_SPEC, out_shape=_ST8,
    )(x_ne, x_ce)

    y0, st0 = pl.pallas_call(
        _p1_kernel, grid=grid,
        in_specs=[_row_spec(128), _CE_SPEC, ab_spec, ab_spec, w0_spec,
                  _CONST8],
        out_specs=(_row_spec(_DOUT), _ACC_SPEC),
        out_shape=(yshape, _ST8),
    )(x_ne, x_ce, _bcast8(alpha), _bcast8(beta), W0, st_d)

    def generic(yp, st, g, b, w):
        return pl.pallas_call(
            _p2_kernel, grid=grid,
            in_specs=[_row_spec(_DOUT), _CONST8, _CONST8, _CONST8, w_spec],
            out_specs=(_row_spec(_DOUT), _ACC_SPEC),
            out_shape=(yshape, _ST8),
        )(yp, st, _bcast8(g), _bcast8(b), w)

    y1, st1 = generic(y0, st0, g0, b0, W1a)
    y2, st2 = generic(y1, st1, g1a, b1a, W2a)

    y3, h1, st3 = pl.pallas_call(
        _p4_kernel, grid=grid,
        in_specs=[_row_spec(_DOUT), _row_spec(_DOUT), _CONST8, _CONST8,
                  _CONST8, _CONST8, _CONST8, _CONST8, w_spec],
        out_specs=(_row_spec(_DOUT), _row_spec(_DOUT), _ACC_SPEC),
        out_shape=(yshape, yshape, _ST8),
    )(y2, y0, st2, _bcast8(g2a), _bcast8(b2a),
      st0, _bcast8(g0), _bcast8(b0), W1b)

    y4, st4 = generic(y3, st3, g1b, b1b, W2b)

    out = pl.pallas_call(
        _p6_kernel, grid=grid,
        in_specs=[_row_spec(_DOUT), _row_spec(_DOUT), _CONST8, _CONST8,
                  _CONST8],
        out_specs=pl.BlockSpec((_QT, _DOUT), lambda i: (i, 0)),
        out_shape=jax.ShapeDtypeStruct((_B * _NQ, _DOUT), jnp.float32),
    )(y4, h1, st4, _bcast8(g2b), _bcast8(b2b))
    return out.reshape(_B, _NQ, _DOUT)


def kernel(x, z, alpha, beta, W0, g0, b0, W1_0, g1_0, b1_0, W2_0, g2_0, b2_0,
           W1_1, g1_1, b1_1, W2_1, g2_1, b2_1):
    u_ce, z_ce = _run_fps(z)

    negd = _run_dist(z_ce, z)
    _, u_ne = lax.top_k(negd, _NG)
    x_ne, x_ce = _run_gather(x, u_ne, u_ce)

    blocks = ((W1_0, g1_0, b1_0, W2_0, g2_0, b2_0),
              (W1_1, g1_1, b1_1, W2_1, g2_1, b2_1))
    x_out = _run_mlp(x_ne, x_ce, alpha, beta, W0, g0, b0, blocks)
    return x_out, z_ce
